# baseline (speedup 1.0000x reference)
import functools

import jax
import jax.numpy as jnp
from jax import lax
from jax.experimental import pallas as pl
from jax.experimental.pallas import tpu as pltpu
from jax.experimental.pallas import tpu_sc as plsc

NUM_BUCKETS = 1000000
NC, NS, LANES = 2, 16, 16
NUM_WORKERS = NC * NS


def _sc_hash_gather(cur, prev, table_wide):
    n = cur.shape[0]
    w = table_wide.shape[1]
    b_per_w = n // NUM_WORKERS
    mesh = plsc.VectorSubcoreMesh(core_axis_name="c", subcore_axis_name="s")

    @functools.partial(
        pl.kernel,
        mesh=mesh,
        out_type=(
            jax.ShapeDtypeStruct((n, w), jnp.float32),
            jax.ShapeDtypeStruct((n,), jnp.int32),
        ),
        compiler_params=pltpu.CompilerParams(use_tc_tiling_on_sc=False),
        scratch_types=[
            pltpu.VMEM((b_per_w,), jnp.int32),
            pltpu.VMEM((b_per_w,), jnp.int32),
            pltpu.VMEM((b_per_w,), jnp.int32),
            pltpu.VMEM((b_per_w,), jnp.int32),
            pltpu.VMEM((b_per_w, w), jnp.float32),
            pltpu.SemaphoreType.DMA,
        ],
    )
    def gather_kernel(cur_hbm, prev_hbm, table_hbm, rows_out_hbm, idx_out_hbm,
                      cur_v, prev_v, idx_v, idx4_v, rows_v, sem):
        wid = (lax.axis_index("s") * jnp.int32(NC)
               + lax.axis_index("c")).astype(jnp.int32)
        base = wid * jnp.int32(b_per_w)
        pltpu.sync_copy(cur_hbm.at[pl.ds(base, b_per_w)], cur_v)
        pltpu.sync_copy(prev_hbm.at[pl.ds(base, b_per_w)], prev_v)

        k10 = jnp.full((LANES,), 10, dtype=jnp.int32)
        k100k = jnp.full((LANES,), 100000, dtype=jnp.int32)
        k4729 = jnp.full((LANES,), 4729, dtype=jnp.int32)
        kmod = jnp.full((LANES,), NUM_BUCKETS, dtype=jnp.int32)
        k2 = jnp.full((LANES,), 2, dtype=jnp.int32)

        @pl.loop(0, b_per_w, step=LANES)
        def _(i):
            p = prev_v[pl.ds(i, LANES)]
            c = cur_v[pl.ds(i, LANES)]
            h = ((p % k10) * k100k + p * k4729 + c) % kmod
            idx_v[pl.ds(i, LANES)] = h
            idx4_v[pl.ds(i, LANES)] = lax.shift_right_logical(h, k2)

        pltpu.async_copy(table_hbm.at[idx4_v], rows_v, sem).wait()
        pltpu.sync_copy(rows_v, rows_out_hbm.at[pl.ds(base, b_per_w)])
        pltpu.sync_copy(idx_v, idx_out_hbm.at[pl.ds(base, b_per_w)])

    return gather_kernel(cur, prev, table_wide)


def _tc_select_project(rows, bidx3, w_t, block_rows=2048):
    n, w = rows.shape
    e, m = w_t.shape
    pack = w // e
    grid = n // block_rows

    def body(g_ref, i_ref, w_ref, o_ref):
        slot = i_ref[0] % pack
        emb = jnp.zeros((block_rows, e), jnp.float32)
        for k in range(pack):
            sel = slot == k
            emb = emb + jnp.where(sel, g_ref[:, k * e:(k + 1) * e], 0.0)
        o_ref[...] = jnp.dot(emb, w_ref[...], preferred_element_type=jnp.float32)

    return pl.pallas_call(
        body,
        out_shape=jax.ShapeDtypeStruct((n, m), jnp.float32),
        grid=(grid,),
        in_specs=[
            pl.BlockSpec((block_rows, w), lambda i: (i, 0)),
            pl.BlockSpec((1, block_rows, 1), lambda i: (i, 0, 0)),
            pl.BlockSpec((e, m), lambda i: (0, 0)),
        ],
        out_specs=pl.BlockSpec((block_rows, m), lambda i: (i, 0)),
    )(rows, bidx3, w_t)


def kernel(input_ids, embed_weight, proj_weight):
    b, s = input_ids.shape
    m = proj_weight.shape[0]
    e = embed_weight.shape[1]
    n = b * s
    block_rows = 2048
    ids32 = input_ids.astype(jnp.int32)
    prev32 = jnp.concatenate([ids32[:, :1], ids32[:, :-1]], axis=1)
    cur = ids32.reshape(-1)
    prev = prev32.reshape(-1)
    table_wide = embed_weight.reshape(-1, 4 * e)
    with jax.enable_x64(False):
        rows, bidx = _sc_hash_gather(cur, prev, table_wide)
        bidx3 = bidx.reshape(n // block_rows, block_rows, 1)
        out = _tc_select_project(rows, bidx3, proj_weight.T, block_rows)
    return out.reshape(b, s, m)


# speedup vs baseline: 1.7509x; 1.7509x over previous
import functools

import jax
import jax.numpy as jnp
from jax import lax
from jax.experimental import pallas as pl
from jax.experimental.pallas import tpu as pltpu
from jax.experimental.pallas import tpu_sc as plsc

NUM_BUCKETS = 1000000
NC, NS, LANES = 2, 16, 16
NUM_WORKERS = NC * NS


CHUNK = 1 << 18


def _tc_pack_rows(table_t, cols_per_blk=8192):
    e, v = table_t.shape
    w = 4 * e
    grid = CHUNK // cols_per_blk
    n_lane_blocks = (v + cols_per_blk - 1) // cols_per_blk

    def body(x0_ref, x1_ref, x2_ref, x3_ref, o_ref):
        for k, x_ref in enumerate((x0_ref, x1_ref, x2_ref, x3_ref)):
            o_ref[:, k * e:(k + 1) * e] = x_ref[...].T

    def in_map(k):
        off = k * grid
        return lambda i: (0, jnp.minimum(i + off, n_lane_blocks - 1))

    return pl.pallas_call(
        body,
        out_shape=jax.ShapeDtypeStruct((CHUNK, w), jnp.float32),
        grid=(grid,),
        in_specs=[pl.BlockSpec((e, cols_per_blk), in_map(k)) for k in range(4)],
        out_specs=pl.BlockSpec((cols_per_blk, w), lambda i: (i, 0)),
    )(table_t, table_t, table_t, table_t)


def _sc_hash_gather(cur, prev, table_wide):
    n = cur.shape[0]
    w = table_wide.shape[1]
    b_per_w = n // NUM_WORKERS
    mesh = plsc.VectorSubcoreMesh(core_axis_name="c", subcore_axis_name="s")

    @functools.partial(
        pl.kernel,
        mesh=mesh,
        out_type=(
            jax.ShapeDtypeStruct((n, w), jnp.float32),
            jax.ShapeDtypeStruct((n,), jnp.int32),
        ),
        compiler_params=pltpu.CompilerParams(use_tc_tiling_on_sc=False),
        scratch_types=[
            pltpu.VMEM((b_per_w,), jnp.int32),
            pltpu.VMEM((b_per_w,), jnp.int32),
            pltpu.VMEM((b_per_w,), jnp.int32),
            pltpu.VMEM((b_per_w,), jnp.int32),
            pltpu.VMEM((b_per_w, w), jnp.float32),
            pltpu.SemaphoreType.DMA,
        ],
    )
    def gather_kernel(cur_hbm, prev_hbm, table_hbm, rows_out_hbm, idx_out_hbm,
                      cur_v, prev_v, idx_v, idx4_v, rows_v, sem):
        wid = (lax.axis_index("s") * jnp.int32(NC)
               + lax.axis_index("c")).astype(jnp.int32)
        base = wid * jnp.int32(b_per_w)
        pltpu.sync_copy(cur_hbm.at[pl.ds(base, b_per_w)], cur_v)
        pltpu.sync_copy(prev_hbm.at[pl.ds(base, b_per_w)], prev_v)

        k10 = jnp.full((LANES,), 10, dtype=jnp.int32)
        k100k = jnp.full((LANES,), 100000, dtype=jnp.int32)
        k4729 = jnp.full((LANES,), 4729, dtype=jnp.int32)
        kmod = jnp.full((LANES,), NUM_BUCKETS, dtype=jnp.int32)
        kmask = jnp.full((LANES,), CHUNK - 1, dtype=jnp.int32)

        @pl.loop(0, b_per_w, step=LANES)
        def _(i):
            p = prev_v[pl.ds(i, LANES)]
            c = cur_v[pl.ds(i, LANES)]
            h = ((p % k10) * k100k + p * k4729 + c) % kmod
            idx_v[pl.ds(i, LANES)] = h
            idx4_v[pl.ds(i, LANES)] = h & kmask

        pltpu.async_copy(table_hbm.at[idx4_v], rows_v, sem).wait()
        pltpu.sync_copy(rows_v, rows_out_hbm.at[pl.ds(base, b_per_w)])
        pltpu.sync_copy(idx_v, idx_out_hbm.at[pl.ds(base, b_per_w)])

    return gather_kernel(cur, prev, table_wide)


def _tc_select_project(rows, bidx3, w_t, block_rows=2048):
    n, w = rows.shape
    e, m = w_t.shape
    pack = w // e
    grid = n // block_rows

    def body(g_ref, i_ref, w_ref, o_ref):
        slot = i_ref[0] >> 18
        emb = jnp.zeros((block_rows, e), jnp.float32)
        for k in range(pack):
            sel = slot == k
            emb = emb + jnp.where(sel, g_ref[:, k * e:(k + 1) * e], 0.0)
        o_ref[...] = jnp.dot(emb, w_ref[...], preferred_element_type=jnp.float32)

    return pl.pallas_call(
        body,
        out_shape=jax.ShapeDtypeStruct((n, m), jnp.float32),
        grid=(grid,),
        in_specs=[
            pl.BlockSpec((block_rows, w), lambda i: (i, 0)),
            pl.BlockSpec((1, block_rows, 1), lambda i: (i, 0, 0)),
            pl.BlockSpec((e, m), lambda i: (0, 0)),
        ],
        out_specs=pl.BlockSpec((block_rows, m), lambda i: (i, 0)),
    )(rows, bidx3, w_t)


def kernel(input_ids, embed_weight, proj_weight):
    b, s = input_ids.shape
    m = proj_weight.shape[0]
    n = b * s
    block_rows = 2048
    ids32 = input_ids.astype(jnp.int32)
    prev32 = jnp.concatenate([ids32[:, :1], ids32[:, :-1]], axis=1)
    cur = ids32.reshape(-1)
    prev = prev32.reshape(-1)
    with jax.enable_x64(False):
        table_wide = _tc_pack_rows(embed_weight.T)
        rows, bidx = _sc_hash_gather(cur, prev, table_wide)
        bidx3 = bidx.reshape(n // block_rows, block_rows, 1)
        out = _tc_select_project(rows, bidx3, proj_weight.T, block_rows)
    return out.reshape(b, s, m)


# speedup vs baseline: 3.6306x; 2.0736x over previous
import functools

import jax
import jax.numpy as jnp
from jax import lax
from jax.experimental import pallas as pl
from jax.experimental.pallas import tpu as pltpu
from jax.experimental.pallas import tpu_sc as plsc

NUM_BUCKETS = 1000000
NC, NS, LANES = 2, 16, 16
NUM_WORKERS = NC * NS


CHUNK = 1 << 18


def _tc_pack_rows(table_t, cols_per_blk=8192):
    e, v = table_t.shape
    w = 4 * e
    grid = CHUNK // cols_per_blk
    n_lane_blocks = (v + cols_per_blk - 1) // cols_per_blk

    def body(x0_ref, x1_ref, x2_ref, x3_ref, o_ref):
        xcat = jnp.concatenate(
            [x0_ref[...], x1_ref[...], x2_ref[...], x3_ref[...]], axis=0)
        o_ref[...] = xcat.T

    def in_map(k):
        off = k * grid
        return lambda i: (0, jnp.minimum(i + off, n_lane_blocks - 1))

    return pl.pallas_call(
        body,
        out_shape=jax.ShapeDtypeStruct((CHUNK, w), jnp.float32),
        grid=(grid,),
        in_specs=[pl.BlockSpec((e, cols_per_blk), in_map(k)) for k in range(4)],
        out_specs=pl.BlockSpec((cols_per_blk, w), lambda i: (i, 0)),
        compiler_params=pltpu.CompilerParams(fuse_transposed_lhs_in_matmul=True),
    )(table_t, table_t, table_t, table_t)


def _sc_hash_gather(cur, prev, table_wide):
    n = cur.shape[0]
    w = table_wide.shape[1]
    b_per_w = n // NUM_WORKERS
    mesh = plsc.VectorSubcoreMesh(core_axis_name="c", subcore_axis_name="s")

    @functools.partial(
        pl.kernel,
        mesh=mesh,
        out_type=(
            jax.ShapeDtypeStruct((n, w), jnp.float32),
            jax.ShapeDtypeStruct((n,), jnp.int32),
        ),
        compiler_params=pltpu.CompilerParams(use_tc_tiling_on_sc=False),
        scratch_types=[
            pltpu.VMEM((b_per_w,), jnp.int32),
            pltpu.VMEM((b_per_w,), jnp.int32),
            pltpu.VMEM((b_per_w,), jnp.int32),
            pltpu.VMEM((b_per_w,), jnp.int32),
            pltpu.VMEM((b_per_w, w), jnp.float32),
            pltpu.SemaphoreType.DMA,
        ],
    )
    def gather_kernel(cur_hbm, prev_hbm, table_hbm, rows_out_hbm, idx_out_hbm,
                      cur_v, prev_v, idx_v, idx4_v, rows_v, sem):
        wid = (lax.axis_index("s") * jnp.int32(NC)
               + lax.axis_index("c")).astype(jnp.int32)
        base = wid * jnp.int32(b_per_w)
        pltpu.sync_copy(cur_hbm.at[pl.ds(base, b_per_w)], cur_v)
        pltpu.sync_copy(prev_hbm.at[pl.ds(base, b_per_w)], prev_v)

        k10 = jnp.full((LANES,), 10, dtype=jnp.int32)
        k100k = jnp.full((LANES,), 100000, dtype=jnp.int32)
        k4729 = jnp.full((LANES,), 4729, dtype=jnp.int32)
        kmod = jnp.full((LANES,), NUM_BUCKETS, dtype=jnp.int32)
        kmask = jnp.full((LANES,), CHUNK - 1, dtype=jnp.int32)

        @pl.loop(0, b_per_w, step=LANES)
        def _(i):
            p = prev_v[pl.ds(i, LANES)]
            c = cur_v[pl.ds(i, LANES)]
            h = ((p % k10) * k100k + p * k4729 + c) % kmod
            idx_v[pl.ds(i, LANES)] = h
            idx4_v[pl.ds(i, LANES)] = h & kmask

        pltpu.async_copy(table_hbm.at[idx4_v], rows_v, sem).wait()
        pltpu.sync_copy(rows_v, rows_out_hbm.at[pl.ds(base, b_per_w)])
        pltpu.sync_copy(idx_v, idx_out_hbm.at[pl.ds(base, b_per_w)])

    return gather_kernel(cur, prev, table_wide)


def _tc_select_project(rows, bidx3, w_t, block_rows=2048):
    n, w = rows.shape
    e, m = w_t.shape
    pack = w // e
    grid = n // block_rows

    def body(g_ref, i_ref, w_ref, o_ref):
        slot = i_ref[0] >> 18
        emb = jnp.zeros((block_rows, e), jnp.float32)
        for k in range(pack):
            sel = slot == k
            emb = emb + jnp.where(sel, g_ref[:, k * e:(k + 1) * e], 0.0)
        o_ref[...] = jnp.dot(emb, w_ref[...], preferred_element_type=jnp.float32)

    return pl.pallas_call(
        body,
        out_shape=jax.ShapeDtypeStruct((n, m), jnp.float32),
        grid=(grid,),
        in_specs=[
            pl.BlockSpec((block_rows, w), lambda i: (i, 0)),
            pl.BlockSpec((1, block_rows, 1), lambda i: (i, 0, 0)),
            pl.BlockSpec((e, m), lambda i: (0, 0)),
        ],
        out_specs=pl.BlockSpec((block_rows, m), lambda i: (i, 0)),
    )(rows, bidx3, w_t)


def kernel(input_ids, embed_weight, proj_weight):
    b, s = input_ids.shape
    m = proj_weight.shape[0]
    n = b * s
    block_rows = 2048
    ids32 = input_ids.astype(jnp.int32)
    prev32 = jnp.concatenate([ids32[:, :1], ids32[:, :-1]], axis=1)
    cur = ids32.reshape(-1)
    prev = prev32.reshape(-1)
    with jax.enable_x64(False):
        table_wide = _tc_pack_rows(embed_weight.T)
        rows, bidx = _sc_hash_gather(cur, prev, table_wide)
        bidx3 = bidx.reshape(n // block_rows, block_rows, 1)
        out = _tc_select_project(rows, bidx3, proj_weight.T, block_rows)
    return out.reshape(b, s, m)


# speedup vs baseline: 3.8339x; 1.0560x over previous
import functools

import jax
import jax.numpy as jnp
from jax import lax
from jax.experimental import pallas as pl
from jax.experimental.pallas import tpu as pltpu
from jax.experimental.pallas import tpu_sc as plsc

NUM_BUCKETS = 1000000
NC, NS, LANES = 2, 16, 16
NUM_WORKERS = NC * NS


CHUNK = 1 << 18


def _tc_pack_rows(table_t, cols_per_blk=8192):
    e, v = table_t.shape
    w = 4 * e
    grid = CHUNK // cols_per_blk
    n_lane_blocks = (v + cols_per_blk - 1) // cols_per_blk

    def body(x0_ref, x1_ref, x2_ref, x3_ref, o_ref):
        xcat = jnp.concatenate(
            [x0_ref[...], x1_ref[...], x2_ref[...], x3_ref[...]], axis=0)
        o_ref[...] = xcat.T

    def in_map(k):
        off = k * grid
        return lambda i: (0, jnp.minimum(i + off, n_lane_blocks - 1))

    return pl.pallas_call(
        body,
        out_shape=jax.ShapeDtypeStruct((CHUNK, w), jnp.float32),
        grid=(grid,),
        in_specs=[pl.BlockSpec((e, cols_per_blk), in_map(k)) for k in range(4)],
        out_specs=pl.BlockSpec((cols_per_blk, w), lambda i: (i, 0)),
        compiler_params=pltpu.CompilerParams(fuse_transposed_lhs_in_matmul=True),
    )(table_t, table_t, table_t, table_t)


def _sc_hash_gather(cur, prev, table_wide):
    n = cur.shape[0]
    w = table_wide.shape[1]
    b_per_w = n // NUM_WORKERS
    mesh = plsc.VectorSubcoreMesh(core_axis_name="c", subcore_axis_name="s")

    @functools.partial(
        pl.kernel,
        mesh=mesh,
        out_type=jax.ShapeDtypeStruct((n, w), jnp.float32),
        compiler_params=pltpu.CompilerParams(use_tc_tiling_on_sc=False,
                                             needs_layout_passes=False),
        scratch_types=[
            pltpu.VMEM((b_per_w,), jnp.int32),
            pltpu.VMEM((b_per_w,), jnp.int32),
            pltpu.VMEM((b_per_w,), jnp.int32),
            pltpu.VMEM((b_per_w,), jnp.int32),
            pltpu.VMEM((b_per_w, w), jnp.float32),
            pltpu.SemaphoreType.DMA,
        ],
    )
    def gather_kernel(cur_hbm, prev_hbm, table_hbm, rows_out_hbm,
                      cur_v, prev_v, idx_v, idx4_v, rows_v, sem):
        wid = (lax.axis_index("s") * jnp.int32(NC)
               + lax.axis_index("c")).astype(jnp.int32)
        base = wid * jnp.int32(b_per_w)
        pltpu.sync_copy(cur_hbm.at[pl.ds(base, b_per_w)], cur_v)
        pltpu.sync_copy(prev_hbm.at[pl.ds(base, b_per_w)], prev_v)

        k10 = jnp.full((LANES,), 10, dtype=jnp.int32)
        k100k = jnp.full((LANES,), 100000, dtype=jnp.int32)
        k4729 = jnp.full((LANES,), 4729, dtype=jnp.int32)
        kmod = jnp.full((LANES,), NUM_BUCKETS, dtype=jnp.int32)
        kmask = jnp.full((LANES,), CHUNK - 1, dtype=jnp.int32)

        @pl.loop(0, b_per_w, step=LANES)
        def _(i):
            p = prev_v[pl.ds(i, LANES)]
            c = cur_v[pl.ds(i, LANES)]
            h = ((p % k10) * k100k + p * k4729 + c) % kmod
            idx_v[pl.ds(i, LANES)] = h
            idx4_v[pl.ds(i, LANES)] = h & kmask

        pltpu.async_copy(table_hbm.at[idx4_v], rows_v, sem).wait()

        iota16 = jnp.arange(LANES, dtype=jnp.int32)
        col0 = jnp.zeros((LANES,), jnp.int32)
        col1 = jnp.ones((LANES,), jnp.int32)
        km2 = jnp.full((LANES,), -2, dtype=jnp.int32)
        k1 = jnp.full((LANES,), 1, dtype=jnp.int32)
        k18 = jnp.full((LANES,), 18, dtype=jnp.int32)
        k19 = jnp.full((LANES,), 19, dtype=jnp.int32)

        @pl.loop(0, b_per_w, step=LANES)
        def _(j):
            tok = iota16 + lax.broadcast_in_dim(j, (LANES,), ())
            h = idx_v[pl.ds(j, LANES)]
            b0 = lax.shift_right_logical(h, k18) & k1
            b1 = lax.shift_right_logical(h, k19) & k1
            v0 = plsc.bitcast(plsc.load_gather(rows_v, [tok, col0]), jnp.int32)
            v1 = plsc.bitcast(plsc.load_gather(rows_v, [tok, col1]), jnp.int32)
            plsc.store_scatter(rows_v, [tok, col0],
                               plsc.bitcast((v0 & km2) | b0, jnp.float32))
            plsc.store_scatter(rows_v, [tok, col1],
                               plsc.bitcast((v1 & km2) | b1, jnp.float32))

        pltpu.sync_copy(rows_v, rows_out_hbm.at[pl.ds(base, b_per_w)])

    return gather_kernel(cur, prev, table_wide)


def _tc_select_project(rows, w_t, block_rows=4096):
    n, w = rows.shape
    e, m = w_t.shape
    pack = w // e
    grid = n // block_rows

    def body(g_ref, w_ref, o_ref):
        g0 = lax.bitcast_convert_type(g_ref[:, 0:1], jnp.int32)
        g1 = lax.bitcast_convert_type(g_ref[:, 1:2], jnp.int32)
        slot = (g0 & 1) | ((g1 & 1) << 1)
        emb = jnp.zeros((block_rows, e), jnp.float32)
        for k in range(pack):
            sel = slot == k
            emb = emb + jnp.where(sel, g_ref[:, k * e:(k + 1) * e], 0.0)
        o_ref[...] = jnp.dot(emb, w_ref[...], preferred_element_type=jnp.float32)

    return pl.pallas_call(
        body,
        out_shape=jax.ShapeDtypeStruct((n, m), jnp.float32),
        grid=(grid,),
        in_specs=[
            pl.BlockSpec((block_rows, w), lambda i: (i, 0)),
            pl.BlockSpec((e, m), lambda i: (0, 0)),
        ],
        out_specs=pl.BlockSpec((block_rows, m), lambda i: (i, 0)),
    )(rows, w_t)


def kernel(input_ids, embed_weight, proj_weight):
    b, s = input_ids.shape
    m = proj_weight.shape[0]
    ids32 = input_ids.astype(jnp.int32)
    prev32 = jnp.concatenate([ids32[:, :1], ids32[:, :-1]], axis=1)
    cur = ids32.reshape(-1)
    prev = prev32.reshape(-1)
    with jax.enable_x64(False):
        table_wide = _tc_pack_rows(embed_weight.T)
        rows = _sc_hash_gather(cur, prev, table_wide)
        out = _tc_select_project(rows, proj_weight.T)
    return out.reshape(b, s, m)


# speedup vs baseline: 3.8976x; 1.0166x over previous
import functools

import jax
import jax.numpy as jnp
from jax import lax
from jax.experimental import pallas as pl
from jax.experimental.pallas import tpu as pltpu
from jax.experimental.pallas import tpu_sc as plsc

NUM_BUCKETS = 1000000
NC, NS, LANES = 2, 16, 16
NUM_WORKERS = NC * NS


CHUNK = 1 << 18


def _tc_pack_rows(table_t, cols_per_blk=16384):
    e, v = table_t.shape
    w = 4 * e
    grid = CHUNK // cols_per_blk
    n_lane_blocks = (v + cols_per_blk - 1) // cols_per_blk

    def body(x0_ref, x1_ref, x2_ref, x3_ref, o_ref):
        xcat = jnp.concatenate(
            [x0_ref[...], x1_ref[...], x2_ref[...], x3_ref[...]], axis=0)
        o_ref[...] = xcat.T

    def in_map(k):
        off = k * grid
        return lambda i: (0, jnp.minimum(i + off, n_lane_blocks - 1))

    return pl.pallas_call(
        body,
        out_shape=jax.ShapeDtypeStruct((CHUNK, w), jnp.float32),
        grid=(grid,),
        in_specs=[pl.BlockSpec((e, cols_per_blk), in_map(k)) for k in range(4)],
        out_specs=pl.BlockSpec((cols_per_blk, w), lambda i: (i, 0)),
        compiler_params=pltpu.CompilerParams(fuse_transposed_lhs_in_matmul=True),
    )(table_t, table_t, table_t, table_t)


def _sc_hash_gather(cur, prev, table_wide):
    n = cur.shape[0]
    w = table_wide.shape[1]
    b_per_w = n // NUM_WORKERS
    mesh = plsc.VectorSubcoreMesh(core_axis_name="c", subcore_axis_name="s")

    @functools.partial(
        pl.kernel,
        mesh=mesh,
        out_type=jax.ShapeDtypeStruct((n, w), jnp.float32),
        compiler_params=pltpu.CompilerParams(use_tc_tiling_on_sc=False,
                                             needs_layout_passes=False),
        scratch_types=[
            pltpu.VMEM((b_per_w,), jnp.int32),
            pltpu.VMEM((b_per_w,), jnp.int32),
            pltpu.VMEM((b_per_w,), jnp.int32),
            pltpu.VMEM((b_per_w,), jnp.int32),
            pltpu.VMEM((b_per_w, w), jnp.float32),
            pltpu.SemaphoreType.DMA,
        ],
    )
    def gather_kernel(cur_hbm, prev_hbm, table_hbm, rows_out_hbm,
                      cur_v, prev_v, idx_v, idx4_v, rows_v, sem):
        wid = (lax.axis_index("s") * jnp.int32(NC)
               + lax.axis_index("c")).astype(jnp.int32)
        base = wid * jnp.int32(b_per_w)
        pltpu.sync_copy(cur_hbm.at[pl.ds(base, b_per_w)], cur_v)
        pltpu.sync_copy(prev_hbm.at[pl.ds(base, b_per_w)], prev_v)

        k10 = jnp.full((LANES,), 10, dtype=jnp.int32)
        k100k = jnp.full((LANES,), 100000, dtype=jnp.int32)
        k4729 = jnp.full((LANES,), 4729, dtype=jnp.int32)
        kmod = jnp.full((LANES,), NUM_BUCKETS, dtype=jnp.int32)
        kmask = jnp.full((LANES,), CHUNK - 1, dtype=jnp.int32)

        @pl.loop(0, b_per_w, step=LANES)
        def _(i):
            p = prev_v[pl.ds(i, LANES)]
            c = cur_v[pl.ds(i, LANES)]
            h = ((p % k10) * k100k + p * k4729 + c) % kmod
            idx_v[pl.ds(i, LANES)] = h
            idx4_v[pl.ds(i, LANES)] = h & kmask

        pltpu.async_copy(table_hbm.at[idx4_v], rows_v, sem).wait()

        iota16 = jnp.arange(LANES, dtype=jnp.int32)
        col0 = jnp.zeros((LANES,), jnp.int32)
        col1 = jnp.ones((LANES,), jnp.int32)
        km2 = jnp.full((LANES,), -2, dtype=jnp.int32)
        k1 = jnp.full((LANES,), 1, dtype=jnp.int32)
        k18 = jnp.full((LANES,), 18, dtype=jnp.int32)
        k19 = jnp.full((LANES,), 19, dtype=jnp.int32)

        @pl.loop(0, b_per_w, step=LANES)
        def _(j):
            tok = iota16 + lax.broadcast_in_dim(j, (LANES,), ())
            h = idx_v[pl.ds(j, LANES)]
            b0 = lax.shift_right_logical(h, k18) & k1
            b1 = lax.shift_right_logical(h, k19) & k1
            v0 = plsc.bitcast(plsc.load_gather(rows_v, [tok, col0]), jnp.int32)
            v1 = plsc.bitcast(plsc.load_gather(rows_v, [tok, col1]), jnp.int32)
            plsc.store_scatter(rows_v, [tok, col0],
                               plsc.bitcast((v0 & km2) | b0, jnp.float32))
            plsc.store_scatter(rows_v, [tok, col1],
                               plsc.bitcast((v1 & km2) | b1, jnp.float32))

        pltpu.sync_copy(rows_v, rows_out_hbm.at[pl.ds(base, b_per_w)])

    return gather_kernel(cur, prev, table_wide)


def _tc_select_project(rows, w_t, block_rows=4096):
    n, w = rows.shape
    e, m = w_t.shape
    pack = w // e
    grid = n // block_rows

    def body(g_ref, w_ref, o_ref):
        g0 = lax.bitcast_convert_type(g_ref[:, 0:1], jnp.int32)
        g1 = lax.bitcast_convert_type(g_ref[:, 1:2], jnp.int32)
        slot = (g0 & 1) | ((g1 & 1) << 1)
        emb = jnp.zeros((block_rows, e), jnp.float32)
        for k in range(pack):
            sel = slot == k
            emb = emb + jnp.where(sel, g_ref[:, k * e:(k + 1) * e], 0.0)
        o_ref[...] = jnp.dot(emb, w_ref[...], preferred_element_type=jnp.float32)

    return pl.pallas_call(
        body,
        out_shape=jax.ShapeDtypeStruct((n, m), jnp.float32),
        grid=(grid,),
        in_specs=[
            pl.BlockSpec((block_rows, w), lambda i: (i, 0)),
            pl.BlockSpec((e, m), lambda i: (0, 0)),
        ],
        out_specs=pl.BlockSpec((block_rows, m), lambda i: (i, 0)),
    )(rows, w_t)


def kernel(input_ids, embed_weight, proj_weight):
    b, s = input_ids.shape
    m = proj_weight.shape[0]
    ids32 = input_ids.astype(jnp.int32)
    prev32 = jnp.concatenate([ids32[:, :1], ids32[:, :-1]], axis=1)
    cur = ids32.reshape(-1)
    prev = prev32.reshape(-1)
    with jax.enable_x64(False):
        table_wide = _tc_pack_rows(embed_weight.T)
        rows = _sc_hash_gather(cur, prev, table_wide)
        out = _tc_select_project(rows, proj_weight.T)
    return out.reshape(b, s, m)
